# baseline (speedup 1.0000x reference)
import functools

import jax
import jax.numpy as jnp
from jax import lax
from jax.experimental import pallas as pl
from jax.experimental.pallas import tpu as pltpu
from jax.experimental.pallas import tpu_sc as plsc

_B = 4096
_F = 26
_CARD = 4000
_D = 64
_NW = 32
_BW = _B // _NW
_GB = 16
_NGRP = _BW // _GB


@functools.lru_cache(maxsize=None)
def _make_gather():
    mesh = plsc.VectorSubcoreMesh(core_axis_name="c", subcore_axis_name="s")

    @functools.partial(
        pl.kernel,
        mesh=mesh,
        out_type=jax.ShapeDtypeStruct((_B, _F, _D), jnp.float32),
        compiler_params=pltpu.CompilerParams(use_tc_tiling_on_sc=False),
        scratch_types=[
            pltpu.VMEM((_BW, _F), jnp.int32),
            pltpu.VMEM((_GB, _F, _D), jnp.float32),
            pltpu.SemaphoreType.DMA,
        ],
    )
    def gather_kernel(idx_hbm, table_hbm, out_hbm, idx_v, rows_v, sem):
        wid = lax.axis_index("s") * 2 + lax.axis_index("c")
        b0 = wid * _BW

        pltpu.sync_copy(idx_hbm.at[pl.ds(b0, _BW)], idx_v)

        lane = lax.iota(jnp.int32, 16)
        c_lo = lane * _CARD
        c_hi = (lane + 10) * _CARD
        hi_mask = lane >= 6

        def adjust(r, carry):
            v0 = idx_v[r, pl.ds(0, 16)]
            idx_v[r, pl.ds(0, 16)] = v0 + c_lo
            v1 = idx_v[r, pl.ds(10, 16)]
            idx_v[r, pl.ds(10, 16)] = jnp.where(hi_mask, v1 + c_hi, v1)
            return carry

        lax.fori_loop(0, _BW, adjust, 0)

        def group(grp, carry):
            copies = [
                pltpu.async_copy(
                    table_hbm.at[idx_v.at[grp * _GB + k]],
                    rows_v.at[k],
                    sem,
                )
                for k in range(_GB)
            ]
            for cp in copies:
                cp.wait()
            pltpu.sync_copy(rows_v, out_hbm.at[pl.ds(b0 + grp * _GB, _GB)])
            return carry

        lax.fori_loop(0, _NGRP, group, 0)

    return gather_kernel


def kernel(inputs, table):
    return _make_gather()(inputs.astype(jnp.int32), table)
